# Initial kernel scaffold; baseline (speedup 1.0000x reference)
import jax
import jax.numpy as jnp
from jax.experimental import pallas as pl


def kernel(edge_segments, distance_image):
    raise NotImplementedError("write your pallas kernel here")


# speedup vs baseline: 82.7983x; 82.7983x over previous
import functools

import jax
import jax.numpy as jnp
import numpy as np
from jax import lax
from jax.experimental import pallas as pl
from jax.experimental.pallas import tpu as pltpu
from jax.experimental.pallas import tpu_sc as plsc

_P = 16
_S = 128
_EXT = 32.0
_H = 512
_W = 512
_M = 4096
_N = _M * _P

_MAGIC = np.float32(1.5 * 2.0**23)
_NW = 32
_PPW = _N // _NW
_G = _PPW // 16
_BIGK = np.int32(2**30)


def _tc_idx_body(epr, epc, nur, nuc, ts, out):
    lpr = epr[...] + ts[...] * nur[...]
    lpc = epc[...] + ts[...] * nuc[...]
    rr = jnp.round(lpr)
    cc = jnp.round(lpc)
    rr = jnp.minimum(jnp.maximum(rr, 0.0), float(_H - 1))
    cc = jnp.minimum(jnp.maximum(cc, 0.0), float(_W - 1))
    out[...] = (rr * float(_W) + cc).astype(jnp.int32)


def _tc_expand(epr, epc, nur, nuc, ts):
    B = 512
    grid = _N // B
    col = pl.BlockSpec((B, 1), lambda i: (i, 0))
    return pl.pallas_call(
        _tc_idx_body,
        grid=(grid,),
        in_specs=[col, col, col, col, pl.BlockSpec((1, _S), lambda i: (0, 0))],
        out_specs=pl.BlockSpec((B, _S), lambda i: (i, 0)),
        out_shape=jax.ShapeDtypeStruct((_N, _S), jnp.int32),
    )(epr, epc, nur, nuc, ts)


def _sc_body(idx_hbm, img_hbm, pk_hbm, ts_hbm,
             dist_hbm, psq_hbm, pw_hbm,
             idxbuf, valbuf, epbuf, ribuf, refbuf, tsbuf, dbuf, accbuf,
             sem, sem2):
    wid = lax.axis_index("s") * 2 + lax.axis_index("c")
    pltpu.sync_copy(ts_hbm, tsbuf)
    lanebase = lax.iota(jnp.int32, 16) * _S
    lane16 = lax.iota(jnp.int32, 16)

    def group_body(g, carry):
        acc_sq, acc_w = carry
        base = wid * _PPW + g * 16
        pltpu.sync_copy(idx_hbm.at[pl.ds(base * _S, 16 * _S)], idxbuf)
        pltpu.sync_copy(pk_hbm.at[pl.ds(base * 4, 64)], epbuf)
        epr = epbuf[pl.ds(0, 16)]
        epc = epbuf[pl.ds(16, 16)]
        nur = epbuf[pl.ds(32, 16)]
        nuc = epbuf[pl.ds(48, 16)]
        rr = jnp.minimum(jnp.maximum((epr + _MAGIC) - _MAGIC, 0.0), float(_H - 1))
        cc = jnp.minimum(jnp.maximum((epc + _MAGIC) - _MAGIC, 0.0), float(_W - 1))
        ribuf[...] = (rr * float(_W) + cc).astype(jnp.int32)
        h2 = pltpu.async_copy(img_hbm.at[ribuf], refbuf, sem2)
        hs = [pltpu.async_copy(img_hbm.at[idxbuf.at[pl.ds(j * _S, _S)]],
                               valbuf.at[pl.ds(j * _S, _S)], sem)
              for j in range(16)]
        for h in hs:
            h.wait()
        h2.wait()
        refv = refbuf[...]

        v0 = plsc.load_gather(valbuf, [lanebase])

        def mbody(s, c2):
            prev, cur, kmin = c2
            sp1 = jnp.minimum(s + 1, _S - 1)
            nxt = plsc.load_gather(valbuf, [lanebase + sp1])
            ismin = (cur <= prev) & (cur <= nxt)
            ks = jnp.abs(2 * s - (_S - 1)) * _S + s
            kvec = jnp.where(ismin, jnp.broadcast_to(ks, (16,)),
                             jnp.broadcast_to(_BIGK, (16,)))
            return (cur, nxt, jnp.minimum(kmin, kvec))

        _, _, kmin = lax.fori_loop(
            0, _S, mbody, (v0, v0, jnp.broadcast_to(_BIGK, (16,))))

        s_star = jnp.bitwise_and(kmin, _S - 1)
        w = jnp.where(kmin < _BIGK, 1.0, 0.0).astype(jnp.float32)
        vstar = plsc.load_gather(valbuf, [lanebase + s_star])
        tstar = plsc.load_gather(tsbuf, [s_star])
        dr = ((epr + tstar * nur) - epr) * w
        dc = ((epc + tstar * nuc) - epc) * w
        plsc.store_scatter(dbuf, [lane16 * 2], dr)
        plsc.store_scatter(dbuf, [lane16 * 2 + 1], dc)
        pltpu.sync_copy(dbuf, dist_hbm.at[pl.ds(base * 2, 32)])
        d = refv - vstar
        return (acc_sq + w * d * d, acc_w + w)

    acc_sq, acc_w = lax.fori_loop(
        0, _G, group_body,
        (jnp.zeros((16,), jnp.float32), jnp.zeros((16,), jnp.float32)))
    accbuf[...] = acc_sq
    pltpu.sync_copy(accbuf, psq_hbm.at[pl.ds(wid * 16, 16)])
    accbuf[...] = acc_w
    pltpu.sync_copy(accbuf, pw_hbm.at[pl.ds(wid * 16, 16)])


@functools.lru_cache(maxsize=1)
def _sc_kernel():
    @functools.partial(
        pl.kernel,
        mesh=plsc.VectorSubcoreMesh(core_axis_name="c", subcore_axis_name="s"),
        compiler_params=pltpu.CompilerParams(needs_layout_passes=False),
        out_type=[
            jax.ShapeDtypeStruct((_N * 2,), jnp.float32),
            jax.ShapeDtypeStruct((_NW * 16,), jnp.float32),
            jax.ShapeDtypeStruct((_NW * 16,), jnp.float32),
        ],
        scratch_types=[
            pltpu.VMEM((16 * _S,), jnp.int32),
            pltpu.VMEM((16 * _S,), jnp.float32),
            pltpu.VMEM((64,), jnp.float32),
            pltpu.VMEM((16,), jnp.int32),
            pltpu.VMEM((16,), jnp.float32),
            pltpu.VMEM((_S,), jnp.float32),
            pltpu.VMEM((32,), jnp.float32),
            pltpu.VMEM((16,), jnp.float32),
            pltpu.SemaphoreType.DMA,
            pltpu.SemaphoreType.DMA,
        ],
    )
    def sck(idx_hbm, img_hbm, pk_hbm, ts_hbm,
            dist_hbm, psq_hbm, pw_hbm, *scratch):
        _sc_body(idx_hbm, img_hbm, pk_hbm, ts_hbm,
                 dist_hbm, psq_hbm, pw_hbm, *scratch)
    return sck


def kernel(edge_segments, distance_image):
    img = distance_image[0, 0]
    start = edge_segments[:, 0, :]
    end = edge_segments[:, 1, :]
    t = jnp.linspace(0.0, 1.0, _P + 2)[1:-1]
    points = (1.0 - t)[None, :, None] * start[:, None, :] + t[None, :, None] * end[:, None, :]
    direction = end - start
    normals = jnp.stack([-direction[:, 1], direction[:, 0]], axis=-1)
    edge_points = points.reshape(-1, 2)
    edge_normals = jnp.broadcast_to(
        normals[:, None, :], (normals.shape[0], _P, 2)).reshape(-1, 2)
    n_unit = edge_normals / (jnp.linalg.norm(edge_normals, axis=-1, keepdims=True) + 1e-8)
    ts = jnp.linspace(-_EXT, _EXT, _S)

    epr = edge_points[:, 0:1]
    epc = edge_points[:, 1:2]
    nur = n_unit[:, 0:1]
    nuc = n_unit[:, 1:2]
    idx = _tc_expand(epr, epc, nur, nuc, ts.reshape(1, _S))

    pk = jnp.stack([edge_points[:, 0], edge_points[:, 1],
                    n_unit[:, 0], n_unit[:, 1]], axis=0)
    pk = pk.reshape(4, _N // 16, 16).transpose(1, 0, 2).reshape(-1)
    dist, psq, pw = _sc_kernel()(idx.reshape(-1), img.reshape(-1), pk,
                                 ts.astype(jnp.float32))
    losses = jnp.sum(psq) / jnp.maximum(jnp.sum(pw), 1.0)
    return losses, dist.reshape(_N, 2)


# speedup vs baseline: 129.1495x; 1.5598x over previous
import functools

import jax
import jax.numpy as jnp
import numpy as np
from jax import lax
from jax.experimental import pallas as pl
from jax.experimental.pallas import tpu as pltpu
from jax.experimental.pallas import tpu_sc as plsc

_P = 16
_S = 128
_EXT = 32.0
_H = 512
_W = 512
_M = 4096
_N = _M * _P

_MAGIC = np.float32(1.5 * 2.0**23)
_NW = 32
_PPW = _N // _NW
_G = _PPW // 16
_BIGK = np.int32(2**30)


def _tc_idx_body(epr, epc, nur, nuc, ts, out):
    lpr = epr[...] + ts[...] * nur[...]
    lpc = epc[...] + ts[...] * nuc[...]
    rr = jnp.round(lpr)
    cc = jnp.round(lpc)
    rr = jnp.minimum(jnp.maximum(rr, 0.0), float(_H - 1))
    cc = jnp.minimum(jnp.maximum(cc, 0.0), float(_W - 1))
    out[...] = (rr * float(_W) + cc).astype(jnp.int32)


def _tc_expand(epr, epc, nur, nuc, ts):
    B = 512
    grid = _N // B
    col = pl.BlockSpec((B, 1), lambda i: (i, 0))
    return pl.pallas_call(
        _tc_idx_body,
        grid=(grid,),
        in_specs=[col, col, col, col, pl.BlockSpec((1, _S), lambda i: (0, 0))],
        out_specs=pl.BlockSpec((B, _S), lambda i: (i, 0)),
        out_shape=jax.ShapeDtypeStruct((_N, _S), jnp.int32),
    )(epr, epc, nur, nuc, ts)


_SG = 16
_SUB = 8
_CHI = _SUB * 16 * _S
_CHP = _SUB * 16 * 4
_PV = 136


def _sc_body(idx_hbm, img_hbm, pk_hbm, ts_hbm,
             dist_hbm, psq_hbm, pw_hbm,
             idx0, idx1, pk0, pk1, val0, val1,
             ribuf, refbuf, tsbuf, dbuf, accbuf,
             csem0, csem1, gsem0, gsem1, rsem):
    wid = lax.axis_index("s") * 2 + lax.axis_index("c")
    pltpu.sync_copy(ts_hbm, tsbuf)
    lane16 = lax.iota(jnp.int32, 16)
    lb = lane16 * _PV
    bigk = jnp.broadcast_to(_BIGK, (16,))

    def issue_chunk(sg, idxb, pkb, sem):
        pltpu.async_copy(idx_hbm.at[pl.ds(wid * _PPW * _S + sg * _CHI, _CHI)],
                         idxb, sem)
        pltpu.async_copy(pk_hbm.at[pl.ds(wid * _PPW * 4 + sg * _CHP, _CHP)],
                         pkb, sem)

    def drain_chunk(idxb, pkb, sem):
        pltpu.make_async_copy(idx_hbm.at[pl.ds(0, _CHI)], idxb, sem).wait()
        pltpu.make_async_copy(pk_hbm.at[pl.ds(0, _CHP)], pkb, sem).wait()

    def issue_g(sub, vb, idxb, sem):
        for j in range(16):
            pltpu.async_copy(
                img_hbm.at[idxb.at[pl.ds((sub * 16 + j) * _S, _S)]],
                vb.at[pl.ds(j * _PV, _S)], sem)

    def drain_g(vb, sem):
        pltpu.make_async_copy(img_hbm.at[pl.ds(0, 16 * _S)],
                              vb.at[pl.ds(0, 16 * _S)], sem).wait()

    def minima(vb):
        prev0 = plsc.load_gather(vb, [lb])

        def it(i, c2):
            prev, kmin = c2
            s0 = i * 8
            vv = [plsc.load_gather(vb, [lb + (s0 + k)]) for k in range(8)]
            vv.append(plsc.load_gather(vb, [lb + jnp.minimum(s0 + 8, _S - 1)]))
            for k in range(8):
                s = s0 + k
                pk_ = prev if k == 0 else vv[k - 1]
                ismin = (vv[k] <= pk_) & (vv[k] <= vv[k + 1])
                ks = jnp.abs(2 * s - (_S - 1)) * _S + s
                kmin = jnp.minimum(
                    kmin, jnp.where(ismin, jnp.broadcast_to(ks, (16,)), bigk))
            return (vv[7], kmin)

        _, kmin = lax.fori_loop(0, _S // 8, it, (prev0, bigk))
        return kmin

    def process(sg, idxb, pkb, carry):
        acc_sq, acc_w = carry
        for sub in range(_SUB):
            epr = pkb[pl.ds(sub * 64, 16)]
            epc = pkb[pl.ds(sub * 64 + 16, 16)]
            rr = jnp.minimum(jnp.maximum((epr + _MAGIC) - _MAGIC, 0.0),
                             float(_H - 1))
            cc = jnp.minimum(jnp.maximum((epc + _MAGIC) - _MAGIC, 0.0),
                             float(_W - 1))
            ribuf[pl.ds(sub * 16, 16)] = (rr * float(_W) + cc).astype(jnp.int32)
        pltpu.async_copy(img_hbm.at[ribuf], refbuf, rsem)
        issue_g(0, val0, idxb, gsem0)
        issue_g(1, val1, idxb, gsem1)
        for sub in range(_SUB):
            vb = val0 if sub % 2 == 0 else val1
            gs = gsem0 if sub % 2 == 0 else gsem1
            drain_g(vb, gs)
            if sub == 0:
                pltpu.make_async_copy(img_hbm.at[pl.ds(0, 128)], refbuf,
                                      rsem).wait()
            kmin = minima(vb)
            s_star = jnp.bitwise_and(kmin, _S - 1)
            w = jnp.where(kmin < bigk, 1.0, 0.0).astype(jnp.float32)
            vstar = plsc.load_gather(vb, [lb + s_star])
            tstar = plsc.load_gather(tsbuf, [s_star])
            epr = pkb[pl.ds(sub * 64, 16)]
            epc = pkb[pl.ds(sub * 64 + 16, 16)]
            nur = pkb[pl.ds(sub * 64 + 32, 16)]
            nuc = pkb[pl.ds(sub * 64 + 48, 16)]
            dr = ((epr + tstar * nur) - epr) * w
            dc = ((epc + tstar * nuc) - epc) * w
            plsc.store_scatter(dbuf, [lane16 * 2 + sub * 32], dr)
            plsc.store_scatter(dbuf, [lane16 * 2 + (sub * 32 + 1)], dc)
            refv = refbuf[pl.ds(sub * 16, 16)]
            d = refv - vstar
            acc_sq = acc_sq + w * d * d
            acc_w = acc_w + w
            if sub + 2 < _SUB:
                issue_g(sub + 2, vb, idxb, gs)
        pltpu.sync_copy(dbuf,
                        dist_hbm.at[pl.ds(wid * _PPW * 2 + sg * 256, 256)])
        return (acc_sq, acc_w)

    issue_chunk(0, idx0, pk0, csem0)
    last = jnp.int32(_SG - 1)

    def outer(o, carry):
        sg = o * 2
        issue_chunk(jnp.minimum(sg + 1, last), idx1, pk1, csem1)
        drain_chunk(idx0, pk0, csem0)
        carry = process(sg, idx0, pk0, carry)
        issue_chunk(jnp.minimum(sg + 2, last), idx0, pk0, csem0)
        drain_chunk(idx1, pk1, csem1)
        carry = process(sg + 1, idx1, pk1, carry)
        return carry

    acc_sq, acc_w = lax.fori_loop(
        0, _SG // 2, outer,
        (jnp.zeros((16,), jnp.float32), jnp.zeros((16,), jnp.float32)))
    drain_chunk(idx0, pk0, csem0)
    accbuf[...] = acc_sq
    pltpu.sync_copy(accbuf, psq_hbm.at[pl.ds(wid * 16, 16)])
    accbuf[...] = acc_w
    pltpu.sync_copy(accbuf, pw_hbm.at[pl.ds(wid * 16, 16)])


@functools.lru_cache(maxsize=1)
def _sc_kernel():
    @functools.partial(
        pl.kernel,
        mesh=plsc.VectorSubcoreMesh(core_axis_name="c", subcore_axis_name="s"),
        compiler_params=pltpu.CompilerParams(needs_layout_passes=False),
        out_type=[
            jax.ShapeDtypeStruct((_N * 2,), jnp.float32),
            jax.ShapeDtypeStruct((_NW * 16,), jnp.float32),
            jax.ShapeDtypeStruct((_NW * 16,), jnp.float32),
        ],
        scratch_types=[
            pltpu.VMEM((_CHI,), jnp.int32),
            pltpu.VMEM((_CHI,), jnp.int32),
            pltpu.VMEM((_CHP,), jnp.float32),
            pltpu.VMEM((_CHP,), jnp.float32),
            pltpu.VMEM((16 * _PV,), jnp.float32),
            pltpu.VMEM((16 * _PV,), jnp.float32),
            pltpu.VMEM((128,), jnp.int32),
            pltpu.VMEM((128,), jnp.float32),
            pltpu.VMEM((_S,), jnp.float32),
            pltpu.VMEM((256,), jnp.float32),
            pltpu.VMEM((16,), jnp.float32),
            pltpu.SemaphoreType.DMA,
            pltpu.SemaphoreType.DMA,
            pltpu.SemaphoreType.DMA,
            pltpu.SemaphoreType.DMA,
            pltpu.SemaphoreType.DMA,
        ],
    )
    def sck(idx_hbm, img_hbm, pk_hbm, ts_hbm,
            dist_hbm, psq_hbm, pw_hbm, *scratch):
        _sc_body(idx_hbm, img_hbm, pk_hbm, ts_hbm,
                 dist_hbm, psq_hbm, pw_hbm, *scratch)
    return sck


def kernel(edge_segments, distance_image):
    img = distance_image[0, 0]
    start = edge_segments[:, 0, :]
    end = edge_segments[:, 1, :]
    t = jnp.linspace(0.0, 1.0, _P + 2)[1:-1]
    points = (1.0 - t)[None, :, None] * start[:, None, :] + t[None, :, None] * end[:, None, :]
    direction = end - start
    normals = jnp.stack([-direction[:, 1], direction[:, 0]], axis=-1)
    edge_points = points.reshape(-1, 2)
    edge_normals = jnp.broadcast_to(
        normals[:, None, :], (normals.shape[0], _P, 2)).reshape(-1, 2)
    n_unit = edge_normals / (jnp.linalg.norm(edge_normals, axis=-1, keepdims=True) + 1e-8)
    ts = jnp.linspace(-_EXT, _EXT, _S)

    epr = edge_points[:, 0:1]
    epc = edge_points[:, 1:2]
    nur = n_unit[:, 0:1]
    nuc = n_unit[:, 1:2]
    idx = _tc_expand(epr, epc, nur, nuc, ts.reshape(1, _S))

    pk = jnp.stack([edge_points[:, 0], edge_points[:, 1],
                    n_unit[:, 0], n_unit[:, 1]], axis=0)
    pk = pk.reshape(4, _N // 16, 16).transpose(1, 0, 2).reshape(-1)
    dist, psq, pw = _sc_kernel()(idx.reshape(-1), img.reshape(-1), pk,
                                 ts.astype(jnp.float32))
    losses = jnp.sum(psq) / jnp.maximum(jnp.sum(pw), 1.0)
    return losses, dist.reshape(_N, 2)


# speedup vs baseline: 177.6655x; 1.3757x over previous
import functools

import jax
import jax.numpy as jnp
import numpy as np
from jax import lax
from jax.experimental import pallas as pl
from jax.experimental.pallas import tpu as pltpu
from jax.experimental.pallas import tpu_sc as plsc

_P = 16
_S = 128
_EXT = 32.0
_H = 512
_W = 512
_M = 4096
_N = _M * _P

_MAGIC = np.float32(1.5 * 2.0**23)
_NW = 32
_PPW = _N // _NW
_SG = 16
_SUB = 8
_BIGK = np.int32(2**30)


def _tc_idx_body(epr, epc, nur, nuc, ts, out):
    lpr = epr[...] + ts[...] * nur[...]
    lpc = epc[...] + ts[...] * nuc[...]
    rr = jnp.round(lpr)
    cc = jnp.round(lpc)
    rr = jnp.minimum(jnp.maximum(rr, 0.0), float(_H - 1))
    cc = jnp.minimum(jnp.maximum(cc, 0.0), float(_W - 1))
    out[...] = (rr * float(_W) + cc).astype(jnp.int32)


def _tc_expand(epr, epc, nur, nuc, ts):
    B = 512
    grid = _N // B
    row = pl.BlockSpec((1, B), lambda i: (0, i))
    return pl.pallas_call(
        _tc_idx_body,
        grid=(grid,),
        in_specs=[row, row, row, row, pl.BlockSpec((_S, 1), lambda i: (0, 0))],
        out_specs=pl.BlockSpec((_S, B), lambda i: (0, i)),
        out_shape=jax.ShapeDtypeStruct((_S, _N), jnp.int32),
    )(epr, epc, nur, nuc, ts)


def _sc_body(idxt_hbm, img_hbm, pk_hbm, ts_hbm,
             dist_hbm, psq_hbm, pw_hbm,
             it0, it1, pk0, pk1, vt0, vt1,
             ribuf, refbuf, tsbuf, dbuf, accbuf,
             csem0, csem1, gsem0, gsem1, rsem):
    wid = lax.axis_index("s") * 2 + lax.axis_index("c")
    pltpu.sync_copy(ts_hbm, tsbuf)
    lane16 = lax.iota(jnp.int32, 16)
    bigk = jnp.broadcast_to(_BIGK, (16,))
    last = jnp.int32(_SG - 1)

    def issue_chunk(sg, itb, pkb, sem):
        pltpu.async_copy(
            idxt_hbm.at[:, pl.ds(wid * _PPW + sg * 128, 128)], itb, sem)
        pltpu.async_copy(
            pk_hbm.at[pl.ds(wid * _PPW * 4 + sg * 512, 512)], pkb, sem)

    def drain_chunk(itb, pkb, sem):
        pltpu.make_async_copy(idxt_hbm.at[:, pl.ds(0, 128)], itb, sem).wait()
        pltpu.make_async_copy(pk_hbm.at[pl.ds(0, 512)], pkb, sem).wait()

    def issue_rows(itb, vtb, sem):
        def it(i, _):
            for k in range(8):
                r = i * 8 + k
                pltpu.async_copy(img_hbm.at[itb.at[r]],
                                 vtb.at[pl.ds(r * 128, 128)], sem)
            return 0
        lax.fori_loop(0, _S // 8, it, 0)

    def drain_rows(vtb, sem):
        pltpu.make_async_copy(img_hbm.at[pl.ds(0, _S * 128)], vtb, sem).wait()

    def process(sg, pkb, vtb, carry):
        acc_sq, acc_w = carry
        for sub in range(_SUB):
            epr = pkb[pl.ds(sub * 64, 16)]
            epc = pkb[pl.ds(sub * 64 + 16, 16)]
            rr = jnp.minimum(jnp.maximum((epr + _MAGIC) - _MAGIC, 0.0),
                             float(_H - 1))
            cc = jnp.minimum(jnp.maximum((epc + _MAGIC) - _MAGIC, 0.0),
                             float(_W - 1))
            ribuf[pl.ds(sub * 16, 16)] = (rr * float(_W) + cc).astype(jnp.int32)
        pltpu.async_copy(img_hbm.at[ribuf], refbuf, rsem)
        for sub in range(_SUB):
            sb = sub * 16
            prev0 = vtb[pl.ds(sb, 16)]

            def it(i, c2, _sb=sb):
                prev, kmin = c2
                s0 = i * 8
                vv = [vtb[pl.ds((s0 + k) * 128 + _sb, 16)] for k in range(8)]
                vv.append(vtb[pl.ds(jnp.minimum(s0 + 8, _S - 1) * 128 + _sb,
                                    16)])
                for k in range(8):
                    s = s0 + k
                    pk_ = prev if k == 0 else vv[k - 1]
                    ismin = (vv[k] <= pk_) & (vv[k] <= vv[k + 1])
                    ks = jnp.abs(2 * s - (_S - 1)) * _S + s
                    kmin = jnp.minimum(
                        kmin,
                        jnp.where(ismin, jnp.broadcast_to(ks, (16,)), bigk))
                return (vv[7], kmin)

            _, kmin = lax.fori_loop(0, _S // 8, it, (prev0, bigk))
            if sub == 0:
                pltpu.make_async_copy(img_hbm.at[pl.ds(0, 128)], refbuf,
                                      rsem).wait()
            s_star = jnp.bitwise_and(kmin, _S - 1)
            w = jnp.where(kmin < bigk, 1.0, 0.0).astype(jnp.float32)
            vstar = plsc.load_gather(vtb, [s_star * 128 + (lane16 + sb)])
            tstar = plsc.load_gather(tsbuf, [s_star])
            epr = pkb[pl.ds(sub * 64, 16)]
            epc = pkb[pl.ds(sub * 64 + 16, 16)]
            nur = pkb[pl.ds(sub * 64 + 32, 16)]
            nuc = pkb[pl.ds(sub * 64 + 48, 16)]
            dr = ((epr + tstar * nur) - epr) * w
            dc = ((epc + tstar * nuc) - epc) * w
            plsc.store_scatter(dbuf, [lane16 * 2 + sub * 32], dr)
            plsc.store_scatter(dbuf, [lane16 * 2 + (sub * 32 + 1)], dc)
            refv = refbuf[pl.ds(sub * 16, 16)]
            d = refv - vstar
            acc_sq = acc_sq + w * d * d
            acc_w = acc_w + w
        pltpu.sync_copy(dbuf,
                        dist_hbm.at[pl.ds(wid * _PPW * 2 + sg * 256, 256)])
        return (acc_sq, acc_w)

    issue_chunk(0, it0, pk0, csem0)
    drain_chunk(it0, pk0, csem0)
    issue_rows(it0, vt0, gsem0)
    issue_chunk(jnp.int32(1), it1, pk1, csem1)

    def outer(o, carry):
        sg = o * 2
        drain_chunk(it1, pk1, csem1)
        drain_rows(vt0, gsem0)
        issue_rows(it1, vt1, gsem1)
        carry = process(sg, pk0, vt0, carry)
        issue_chunk(jnp.minimum(sg + 2, last), it0, pk0, csem0)
        drain_chunk(it0, pk0, csem0)
        drain_rows(vt1, gsem1)
        issue_rows(it0, vt0, gsem0)
        carry = process(sg + 1, pk1, vt1, carry)
        issue_chunk(jnp.minimum(sg + 3, last), it1, pk1, csem1)
        return carry

    acc_sq, acc_w = lax.fori_loop(
        0, _SG // 2, outer,
        (jnp.zeros((16,), jnp.float32), jnp.zeros((16,), jnp.float32)))
    drain_rows(vt0, gsem0)
    drain_chunk(it1, pk1, csem1)
    accbuf[...] = acc_sq
    pltpu.sync_copy(accbuf, psq_hbm.at[pl.ds(wid * 16, 16)])
    accbuf[...] = acc_w
    pltpu.sync_copy(accbuf, pw_hbm.at[pl.ds(wid * 16, 16)])


@functools.lru_cache(maxsize=1)
def _sc_kernel():
    @functools.partial(
        pl.kernel,
        mesh=plsc.VectorSubcoreMesh(core_axis_name="c", subcore_axis_name="s"),
        compiler_params=pltpu.CompilerParams(needs_layout_passes=False),
        out_type=[
            jax.ShapeDtypeStruct((_N * 2,), jnp.float32),
            jax.ShapeDtypeStruct((_NW * 16,), jnp.float32),
            jax.ShapeDtypeStruct((_NW * 16,), jnp.float32),
        ],
        scratch_types=[
            pltpu.VMEM((_S, 128), jnp.int32),
            pltpu.VMEM((_S, 128), jnp.int32),
            pltpu.VMEM((512,), jnp.float32),
            pltpu.VMEM((512,), jnp.float32),
            pltpu.VMEM((_S * 128,), jnp.float32),
            pltpu.VMEM((_S * 128,), jnp.float32),
            pltpu.VMEM((128,), jnp.int32),
            pltpu.VMEM((128,), jnp.float32),
            pltpu.VMEM((_S,), jnp.float32),
            pltpu.VMEM((256,), jnp.float32),
            pltpu.VMEM((16,), jnp.float32),
            pltpu.SemaphoreType.DMA,
            pltpu.SemaphoreType.DMA,
            pltpu.SemaphoreType.DMA,
            pltpu.SemaphoreType.DMA,
            pltpu.SemaphoreType.DMA,
        ],
    )
    def sck(idxt_hbm, img_hbm, pk_hbm, ts_hbm,
            dist_hbm, psq_hbm, pw_hbm, *scratch):
        _sc_body(idxt_hbm, img_hbm, pk_hbm, ts_hbm,
                 dist_hbm, psq_hbm, pw_hbm, *scratch)
    return sck


def kernel(edge_segments, distance_image):
    img = distance_image[0, 0]
    start = edge_segments[:, 0, :]
    end = edge_segments[:, 1, :]
    direction = end - start
    normals = jnp.stack([-direction[:, 1], direction[:, 0]], axis=-1)
    n_unit = normals / (jnp.linalg.norm(normals, axis=-1, keepdims=True) + 1e-8)
    t = jnp.linspace(0.0, 1.0, _P + 2)[1:-1]
    ts = jnp.linspace(-_EXT, _EXT, _S).astype(jnp.float32)

    sr = edge_segments[:, 0, 0]
    sc_ = edge_segments[:, 0, 1]
    er = edge_segments[:, 1, 0]
    ec = edge_segments[:, 1, 1]
    epr_mp = (1.0 - t)[None, :] * sr[:, None] + t[None, :] * er[:, None]
    epc_mp = (1.0 - t)[None, :] * sc_[:, None] + t[None, :] * ec[:, None]
    nur_mp = jnp.broadcast_to(n_unit[:, 0:1], (_M, _P))
    nuc_mp = jnp.broadcast_to(n_unit[:, 1:2], (_M, _P))

    idxt = _tc_expand(epr_mp.reshape(1, _N), epc_mp.reshape(1, _N),
                      nur_mp.reshape(1, _N), nuc_mp.reshape(1, _N),
                      ts.reshape(_S, 1))

    pk = jnp.concatenate([epr_mp, epc_mp, nur_mp, nuc_mp], axis=1).reshape(-1)
    dist, psq, pw = _sc_kernel()(idxt, img.reshape(-1), pk, ts)
    losses = jnp.sum(psq) / jnp.maximum(jnp.sum(pw), 1.0)
    return losses, dist.reshape(_N, 2)


# speedup vs baseline: 216.0876x; 1.2163x over previous
import functools

import jax
import jax.numpy as jnp
import numpy as np
from jax import lax
from jax.experimental import pallas as pl
from jax.experimental.pallas import tpu as pltpu
from jax.experimental.pallas import tpu_sc as plsc

_P = 16
_S = 128
_EXT = 32.0
_H = 512
_W = 512
_M = 4096
_N = _M * _P

_MAGIC = np.float32(1.5 * 2.0**23)
_NW = 32
_PPW = _N // _NW
_SG = 16
_SUB = 8
_BIGK = np.int32(2**30)


def _sc_body(img_hbm, pk_hbm, ts_hbm,
             dist_hbm, psq_hbm, pw_hbm,
             pk0, pk1, ir0, ir1, vt0, vt1,
             ribuf, refbuf, tsbuf, dbuf, accbuf,
             psem0, psem1, gsem0, gsem1, rsem):
    wid = lax.axis_index("s") * 2 + lax.axis_index("c")
    pltpu.sync_copy(ts_hbm, tsbuf.at[pl.ds(0, _S)])
    lane16 = lax.iota(jnp.int32, 16)
    bigk = jnp.broadcast_to(_BIGK, (16,))
    last = jnp.int32(_SG - 1)

    def issue_pk(sg, pkb, sem):
        pltpu.async_copy(
            pk_hbm.at[pl.ds(wid * _PPW * 4 + sg * 512, 512)], pkb, sem)

    def drain_pk(pkb, sem):
        pltpu.make_async_copy(pk_hbm.at[pl.ds(0, 512)], pkb, sem).wait()

    def _rnd_clamp(x, hi):
        return jnp.minimum(jnp.maximum((x + _MAGIC) - _MAGIC, 0.0), hi)

    def comp_issue_rows(pkb, irb, vtb, sem):
        ep = [(pkb[pl.ds(sub * 64, 16)], pkb[pl.ds(sub * 64 + 16, 16)],
               pkb[pl.ds(sub * 64 + 32, 16)], pkb[pl.ds(sub * 64 + 48, 16)])
              for sub in range(_SUB)]

        def it(i, _):
            tsl = tsbuf[pl.ds(i * 8, 16)]
            for k in range(8):
                r = i * 8 + k
                tsk = jnp.broadcast_to(tsl[k], (16,))
                for sub in range(_SUB):
                    epr, epc, nur, nuc = ep[sub]
                    pr = epr + tsk * nur
                    pc = epc + tsk * nuc
                    rr = _rnd_clamp(pr, float(_H - 1))
                    cc = _rnd_clamp(pc, float(_W - 1))
                    irb[pl.ds(r * 128 + sub * 16, 16)] = (
                        rr * float(_W) + cc).astype(jnp.int32)
                pltpu.async_copy(img_hbm.at[irb.at[pl.ds(r * 128, 128)]],
                                 vtb.at[pl.ds(r * 128, 128)], sem)
            return 0

        lax.fori_loop(0, _S // 8, it, 0)

    def drain_rows(vtb, sem):
        pltpu.make_async_copy(img_hbm.at[pl.ds(0, _S * 128)], vtb, sem).wait()

    def process(sg, pkb, vtb, carry):
        acc_sq, acc_w = carry
        for sub in range(_SUB):
            epr = pkb[pl.ds(sub * 64, 16)]
            epc = pkb[pl.ds(sub * 64 + 16, 16)]
            rr = _rnd_clamp(epr, float(_H - 1))
            cc = _rnd_clamp(epc, float(_W - 1))
            ribuf[pl.ds(sub * 16, 16)] = (rr * float(_W) + cc).astype(jnp.int32)
        pltpu.async_copy(img_hbm.at[ribuf], refbuf, rsem)
        for sub in range(_SUB):
            sb = sub * 16
            prev0 = vtb[pl.ds(sb, 16)]

            def it(i, c2, _sb=sb):
                prev, kmin = c2
                s0 = i * 8
                vv = [vtb[pl.ds((s0 + k) * 128 + _sb, 16)] for k in range(8)]
                vv.append(vtb[pl.ds(jnp.minimum(s0 + 8, _S - 1) * 128 + _sb,
                                    16)])
                for k in range(8):
                    s = s0 + k
                    pk_ = prev if k == 0 else vv[k - 1]
                    ismin = (vv[k] <= pk_) & (vv[k] <= vv[k + 1])
                    ks = jnp.abs(2 * s - (_S - 1)) * _S + s
                    kmin = jnp.minimum(
                        kmin,
                        jnp.where(ismin, jnp.broadcast_to(ks, (16,)), bigk))
                return (vv[7], kmin)

            _, kmin = lax.fori_loop(0, _S // 8, it, (prev0, bigk))
            if sub == 0:
                pltpu.make_async_copy(img_hbm.at[pl.ds(0, 128)], refbuf,
                                      rsem).wait()
            s_star = jnp.bitwise_and(kmin, _S - 1)
            w = jnp.where(kmin < bigk, 1.0, 0.0).astype(jnp.float32)
            vstar = plsc.load_gather(vtb, [s_star * 128 + (lane16 + sb)])
            tstar = plsc.load_gather(tsbuf, [s_star])
            epr = pkb[pl.ds(sub * 64, 16)]
            epc = pkb[pl.ds(sub * 64 + 16, 16)]
            nur = pkb[pl.ds(sub * 64 + 32, 16)]
            nuc = pkb[pl.ds(sub * 64 + 48, 16)]
            dr = ((epr + tstar * nur) - epr) * w
            dc = ((epc + tstar * nuc) - epc) * w
            plsc.store_scatter(dbuf, [lane16 * 2 + sub * 32], dr)
            plsc.store_scatter(dbuf, [lane16 * 2 + (sub * 32 + 1)], dc)
            refv = refbuf[pl.ds(sub * 16, 16)]
            d = refv - vstar
            acc_sq = acc_sq + w * d * d
            acc_w = acc_w + w
        pltpu.sync_copy(dbuf,
                        dist_hbm.at[pl.ds(wid * _PPW * 2 + sg * 256, 256)])
        return (acc_sq, acc_w)

    issue_pk(0, pk0, psem0)
    drain_pk(pk0, psem0)
    comp_issue_rows(pk0, ir0, vt0, gsem0)
    issue_pk(jnp.int32(1), pk1, psem1)

    def outer(o, carry):
        sg = o * 2
        drain_pk(pk1, psem1)
        comp_issue_rows(pk1, ir1, vt1, gsem1)
        drain_rows(vt0, gsem0)
        carry = process(sg, pk0, vt0, carry)
        issue_pk(jnp.minimum(sg + 2, last), pk0, psem0)
        drain_pk(pk0, psem0)
        comp_issue_rows(pk0, ir0, vt0, gsem0)
        drain_rows(vt1, gsem1)
        carry = process(sg + 1, pk1, vt1, carry)
        issue_pk(jnp.minimum(sg + 3, last), pk1, psem1)
        return carry

    acc_sq, acc_w = lax.fori_loop(
        0, _SG // 2, outer,
        (jnp.zeros((16,), jnp.float32), jnp.zeros((16,), jnp.float32)))
    drain_rows(vt0, gsem0)
    drain_pk(pk1, psem1)
    accbuf[...] = acc_sq
    pltpu.sync_copy(accbuf, psq_hbm.at[pl.ds(wid * 16, 16)])
    accbuf[...] = acc_w
    pltpu.sync_copy(accbuf, pw_hbm.at[pl.ds(wid * 16, 16)])


@functools.lru_cache(maxsize=1)
def _sc_kernel():
    @functools.partial(
        pl.kernel,
        mesh=plsc.VectorSubcoreMesh(core_axis_name="c", subcore_axis_name="s"),
        compiler_params=pltpu.CompilerParams(needs_layout_passes=False),
        out_type=[
            jax.ShapeDtypeStruct((_N * 2,), jnp.float32),
            jax.ShapeDtypeStruct((_NW * 16,), jnp.float32),
            jax.ShapeDtypeStruct((_NW * 16,), jnp.float32),
        ],
        scratch_types=[
            pltpu.VMEM((512,), jnp.float32),
            pltpu.VMEM((512,), jnp.float32),
            pltpu.VMEM((_S * 128,), jnp.int32),
            pltpu.VMEM((_S * 128,), jnp.int32),
            pltpu.VMEM((_S * 128,), jnp.float32),
            pltpu.VMEM((_S * 128,), jnp.float32),
            pltpu.VMEM((128,), jnp.int32),
            pltpu.VMEM((128,), jnp.float32),
            pltpu.VMEM((_S + 16,), jnp.float32),
            pltpu.VMEM((256,), jnp.float32),
            pltpu.VMEM((16,), jnp.float32),
            pltpu.SemaphoreType.DMA,
            pltpu.SemaphoreType.DMA,
            pltpu.SemaphoreType.DMA,
            pltpu.SemaphoreType.DMA,
            pltpu.SemaphoreType.DMA,
        ],
    )
    def sck(img_hbm, pk_hbm, ts_hbm,
            dist_hbm, psq_hbm, pw_hbm, *scratch):
        _sc_body(img_hbm, pk_hbm, ts_hbm,
                 dist_hbm, psq_hbm, pw_hbm, *scratch)
    return sck


def kernel(edge_segments, distance_image):
    img = distance_image[0, 0]
    start = edge_segments[:, 0, :]
    end = edge_segments[:, 1, :]
    direction = end - start
    normals = jnp.stack([-direction[:, 1], direction[:, 0]], axis=-1)
    n_unit = normals / (jnp.linalg.norm(normals, axis=-1, keepdims=True) + 1e-8)
    t = jnp.linspace(0.0, 1.0, _P + 2)[1:-1]
    ts = jnp.linspace(-_EXT, _EXT, _S).astype(jnp.float32)

    sr = edge_segments[:, 0, 0]
    sc_ = edge_segments[:, 0, 1]
    er = edge_segments[:, 1, 0]
    ec = edge_segments[:, 1, 1]
    epr_mp = (1.0 - t)[None, :] * sr[:, None] + t[None, :] * er[:, None]
    epc_mp = (1.0 - t)[None, :] * sc_[:, None] + t[None, :] * ec[:, None]
    nur_mp = jnp.broadcast_to(n_unit[:, 0:1], (_M, _P))
    nuc_mp = jnp.broadcast_to(n_unit[:, 1:2], (_M, _P))

    pk = jnp.concatenate([epr_mp, epc_mp, nur_mp, nuc_mp], axis=1).reshape(-1)
    dist, psq, pw = _sc_kernel()(img.reshape(-1), pk, ts)
    losses = jnp.sum(psq) / jnp.maximum(jnp.sum(pw), 1.0)
    return losses, dist.reshape(_N, 2)


# speedup vs baseline: 234.0637x; 1.0832x over previous
import functools

import jax
import jax.numpy as jnp
import numpy as np
from jax import lax
from jax.experimental import pallas as pl
from jax.experimental.pallas import tpu as pltpu
from jax.experimental.pallas import tpu_sc as plsc

_P = 16
_S = 128
_EXT = 32.0
_H = 512
_W = 512
_M = 4096
_N = _M * _P

_MAGIC = np.float32(1.5 * 2.0**23)
_NW = 32
_PPW = _N // _NW
_SG = 16
_SUB = 8
_BIGK = np.int32(2**30)


def _sc_body(img_hbm, pk_hbm, ts_hbm,
             dist_hbm, psq_hbm, pw_hbm,
             pk0, pk1, ir0, ir1, vt0, vt1,
             ribuf, refbuf, tsbuf, dbuf, accbuf,
             psem0, psem1, gsem0, gsem1, rsem):
    wid = lax.axis_index("s") * 2 + lax.axis_index("c")
    pltpu.sync_copy(ts_hbm, tsbuf.at[pl.ds(0, _S)])
    lane16 = lax.iota(jnp.int32, 16)
    bigk = jnp.broadcast_to(_BIGK, (16,))
    last = jnp.int32(_SG - 1)

    def issue_pk(sg, pkb, sem):
        pltpu.async_copy(
            pk_hbm.at[pl.ds(wid * _PPW * 4 + sg * 512, 512)], pkb, sem)

    def drain_pk(pkb, sem):
        pltpu.make_async_copy(pk_hbm.at[pl.ds(0, 512)], pkb, sem).wait()

    def _rnd_clamp(x, hi):
        return jnp.minimum(jnp.maximum((x + _MAGIC) - _MAGIC, 0.0), hi)

    def comp_issue_rows(pkb, irb, vtb, sem):
        ep = [(pkb[pl.ds(sub * 64, 16)], pkb[pl.ds(sub * 64 + 16, 16)],
               pkb[pl.ds(sub * 64 + 32, 16)], pkb[pl.ds(sub * 64 + 48, 16)])
              for sub in range(_SUB)]

        def it(i, _):
            tsl = tsbuf[pl.ds(i * 8, 16)]
            for k in range(8):
                r = i * 8 + k
                tsk = jnp.broadcast_to(tsl[k], (16,))
                for sub in range(_SUB):
                    epr, epc, nur, nuc = ep[sub]
                    pr = epr + tsk * nur
                    pc = epc + tsk * nuc
                    rr = _rnd_clamp(pr, float(_H - 1))
                    cc = _rnd_clamp(pc, float(_W - 1))
                    irb[pl.ds(r * 128 + sub * 16, 16)] = (
                        rr * float(_W) + cc).astype(jnp.int32)
                pltpu.async_copy(img_hbm.at[irb.at[pl.ds(r * 128, 128)]],
                                 vtb.at[pl.ds(r * 128, 128)], sem)
            return 0

        lax.fori_loop(0, _S // 8, it, 0)

    def drain_rows(vtb, sem):
        pltpu.make_async_copy(img_hbm.at[pl.ds(0, _S * 128)], vtb, sem).wait()

    def process(sg, pkb, vtb, carry):
        acc_sq, acc_w = carry
        for sub in range(_SUB):
            epr = pkb[pl.ds(sub * 64, 16)]
            epc = pkb[pl.ds(sub * 64 + 16, 16)]
            rr = _rnd_clamp(epr, float(_H - 1))
            cc = _rnd_clamp(epc, float(_W - 1))
            ribuf[pl.ds(sub * 16, 16)] = (rr * float(_W) + cc).astype(jnp.int32)
        pltpu.async_copy(img_hbm.at[ribuf], refbuf, rsem)
        for sub in range(_SUB):
            sb = sub * 16
            prev0 = vtb[pl.ds(sb, 16)]

            def it(i, c2, _sb=sb):
                prev, kmin = c2
                s0 = i * 8
                vv = [vtb[pl.ds((s0 + k) * 128 + _sb, 16)] for k in range(8)]
                vv.append(vtb[pl.ds(jnp.minimum(s0 + 8, _S - 1) * 128 + _sb,
                                    16)])
                for k in range(8):
                    s = s0 + k
                    pk_ = prev if k == 0 else vv[k - 1]
                    ismin = (vv[k] <= pk_) & (vv[k] <= vv[k + 1])
                    ks = jnp.abs(2 * s - (_S - 1)) * _S + s
                    kmin = jnp.minimum(
                        kmin,
                        jnp.where(ismin, jnp.broadcast_to(ks, (16,)), bigk))
                return (vv[7], kmin)

            _, kmin = lax.fori_loop(0, _S // 8, it, (prev0, bigk))
            if sub == 0:
                pltpu.make_async_copy(img_hbm.at[pl.ds(0, 128)], refbuf,
                                      rsem).wait()
            s_star = jnp.bitwise_and(kmin, _S - 1)
            w = jnp.where(kmin < bigk, 1.0, 0.0).astype(jnp.float32)
            vstar = plsc.load_gather(vtb, [s_star * 128 + (lane16 + sb)])
            tstar = plsc.load_gather(tsbuf, [s_star])
            epr = pkb[pl.ds(sub * 64, 16)]
            epc = pkb[pl.ds(sub * 64 + 16, 16)]
            nur = pkb[pl.ds(sub * 64 + 32, 16)]
            nuc = pkb[pl.ds(sub * 64 + 48, 16)]
            dr = ((epr + tstar * nur) - epr) * w
            dc = ((epc + tstar * nuc) - epc) * w
            plsc.store_scatter(dbuf, [lane16 + sub * 16, lane16 * 0], dr)
            plsc.store_scatter(dbuf, [lane16 + sub * 16, lane16 * 0 + 1], dc)
            refv = refbuf[pl.ds(sub * 16, 16)]
            d = refv - vstar
            acc_sq = acc_sq + w * d * d
            acc_w = acc_w + w
        pltpu.sync_copy(dbuf,
                        dist_hbm.at[pl.ds(wid * _PPW + sg * 128, 128), :])
        return (acc_sq, acc_w)

    issue_pk(0, pk0, psem0)
    drain_pk(pk0, psem0)
    comp_issue_rows(pk0, ir0, vt0, gsem0)
    issue_pk(jnp.int32(1), pk1, psem1)

    def outer(o, carry):
        sg = o * 2
        drain_pk(pk1, psem1)
        comp_issue_rows(pk1, ir1, vt1, gsem1)
        drain_rows(vt0, gsem0)
        carry = process(sg, pk0, vt0, carry)
        issue_pk(jnp.minimum(sg + 2, last), pk0, psem0)
        drain_pk(pk0, psem0)
        comp_issue_rows(pk0, ir0, vt0, gsem0)
        drain_rows(vt1, gsem1)
        carry = process(sg + 1, pk1, vt1, carry)
        issue_pk(jnp.minimum(sg + 3, last), pk1, psem1)
        return carry

    acc_sq, acc_w = lax.fori_loop(
        0, _SG // 2, outer,
        (jnp.zeros((16,), jnp.float32), jnp.zeros((16,), jnp.float32)))
    drain_rows(vt0, gsem0)
    drain_pk(pk1, psem1)
    accbuf[...] = acc_sq
    pltpu.sync_copy(accbuf, psq_hbm.at[pl.ds(wid * 16, 16)])
    accbuf[...] = acc_w
    pltpu.sync_copy(accbuf, pw_hbm.at[pl.ds(wid * 16, 16)])


@functools.lru_cache(maxsize=1)
def _sc_kernel():
    @functools.partial(
        pl.kernel,
        mesh=plsc.VectorSubcoreMesh(core_axis_name="c", subcore_axis_name="s"),
        compiler_params=pltpu.CompilerParams(needs_layout_passes=False),
        out_type=[
            jax.ShapeDtypeStruct((_N, 2), jnp.float32),
            jax.ShapeDtypeStruct((_NW * 16,), jnp.float32),
            jax.ShapeDtypeStruct((_NW * 16,), jnp.float32),
        ],
        scratch_types=[
            pltpu.VMEM((512,), jnp.float32),
            pltpu.VMEM((512,), jnp.float32),
            pltpu.VMEM((_S * 128,), jnp.int32),
            pltpu.VMEM((_S * 128,), jnp.int32),
            pltpu.VMEM((_S * 128,), jnp.float32),
            pltpu.VMEM((_S * 128,), jnp.float32),
            pltpu.VMEM((128,), jnp.int32),
            pltpu.VMEM((128,), jnp.float32),
            pltpu.VMEM((_S + 16,), jnp.float32),
            pltpu.VMEM((128, 2), jnp.float32),
            pltpu.VMEM((16,), jnp.float32),
            pltpu.SemaphoreType.DMA,
            pltpu.SemaphoreType.DMA,
            pltpu.SemaphoreType.DMA,
            pltpu.SemaphoreType.DMA,
            pltpu.SemaphoreType.DMA,
        ],
    )
    def sck(img_hbm, pk_hbm, ts_hbm,
            dist_hbm, psq_hbm, pw_hbm, *scratch):
        _sc_body(img_hbm, pk_hbm, ts_hbm,
                 dist_hbm, psq_hbm, pw_hbm, *scratch)
    return sck


def kernel(edge_segments, distance_image):
    img = distance_image[0, 0]
    start = edge_segments[:, 0, :]
    end = edge_segments[:, 1, :]
    direction = end - start
    normals = jnp.stack([-direction[:, 1], direction[:, 0]], axis=-1)
    n_unit = normals / (jnp.linalg.norm(normals, axis=-1, keepdims=True) + 1e-8)
    t = jnp.linspace(0.0, 1.0, _P + 2)[1:-1]
    ts = jnp.linspace(-_EXT, _EXT, _S).astype(jnp.float32)

    sr = edge_segments[:, 0, 0]
    sc_ = edge_segments[:, 0, 1]
    er = edge_segments[:, 1, 0]
    ec = edge_segments[:, 1, 1]
    epr_mp = (1.0 - t)[None, :] * sr[:, None] + t[None, :] * er[:, None]
    epc_mp = (1.0 - t)[None, :] * sc_[:, None] + t[None, :] * ec[:, None]
    nur_mp = jnp.broadcast_to(n_unit[:, 0:1], (_M, _P))
    nuc_mp = jnp.broadcast_to(n_unit[:, 1:2], (_M, _P))

    pk = jnp.concatenate([epr_mp, epc_mp, nur_mp, nuc_mp], axis=1).reshape(-1)
    dist, psq, pw = _sc_kernel()(img.reshape(-1), pk, ts)
    losses = jnp.sum(psq) / jnp.maximum(jnp.sum(pw), 1.0)
    return losses, dist


# speedup vs baseline: 547.0364x; 2.3371x over previous
import functools

import jax
import jax.numpy as jnp
import numpy as np
from jax import lax
from jax.experimental import pallas as pl
from jax.experimental.pallas import tpu as pltpu
from jax.experimental.pallas import tpu_sc as plsc

_P = 16
_S = 128
_EXT = 32.0
_H = 512
_W = 512
_M = 4096
_N = _M * _P

_MAGIC = np.float32(1.5 * 2.0**23)
_NW = 32
_PPW = _N // _NW
_SG = 16
_SUB = 8
_BIGK = np.int32(2**30)


def _sc_body(img_hbm, pk_hbm, ts_hbm,
             dist_hbm, psq_hbm, pw_hbm,
             pk0, pk1, ir0, ir1, vt0, vt1,
             ribuf, refbuf, tsbuf, dbuf, accbuf, imgs,
             psem0, psem1, gsem0, gsem1, rsem):
    sid = lax.axis_index("s")
    wid = sid * 2 + lax.axis_index("c")
    pltpu.sync_copy(ts_hbm, tsbuf.at[pl.ds(0, _S)])

    @pl.when(sid == 0)
    def _stage_image():
        pltpu.sync_copy(img_hbm, imgs)

    plsc.subcore_barrier()
    lane16 = lax.iota(jnp.int32, 16)
    bigk = jnp.broadcast_to(_BIGK, (16,))
    last = jnp.int32(_SG - 1)

    def issue_pk(sg, pkb, sem):
        pltpu.async_copy(
            pk_hbm.at[pl.ds(wid * _PPW * 4 + sg * 512, 512)], pkb, sem)

    def drain_pk(pkb, sem):
        pltpu.make_async_copy(pk_hbm.at[pl.ds(0, 512)], pkb, sem).wait()

    def _rnd_clamp(x, hi):
        return jnp.minimum(jnp.maximum((x + _MAGIC) - _MAGIC, 0.0), hi)

    def comp_issue_rows(pkb, irb, vtb, sem):
        ep = [(pkb[pl.ds(sub * 64, 16)], pkb[pl.ds(sub * 64 + 16, 16)],
               pkb[pl.ds(sub * 64 + 32, 16)], pkb[pl.ds(sub * 64 + 48, 16)])
              for sub in range(_SUB)]

        def it(i, _):
            tsl = tsbuf[pl.ds(i * 8, 16)]
            for k in range(8):
                r = i * 8 + k
                tsk = jnp.broadcast_to(tsl[k], (16,))
                for sub in range(_SUB):
                    epr, epc, nur, nuc = ep[sub]
                    pr = epr + tsk * nur
                    pc = epc + tsk * nuc
                    rr = _rnd_clamp(pr, float(_H - 1))
                    cc = _rnd_clamp(pc, float(_W - 1))
                    irb[pl.ds(r * 128 + sub * 16, 16)] = (
                        rr * float(_W) + cc).astype(jnp.int32)
                pltpu.async_copy(imgs.at[irb.at[pl.ds(r * 128, 128)]],
                                 vtb.at[pl.ds(r * 128, 128)], sem)
            return 0

        lax.fori_loop(0, _S // 8, it, 0)

    def drain_rows(vtb, sem):
        pltpu.make_async_copy(img_hbm.at[pl.ds(0, _S * 128)], vtb, sem).wait()

    def process(sg, pkb, vtb, carry):
        acc_sq, acc_w = carry
        for sub in range(_SUB):
            epr = pkb[pl.ds(sub * 64, 16)]
            epc = pkb[pl.ds(sub * 64 + 16, 16)]
            rr = _rnd_clamp(epr, float(_H - 1))
            cc = _rnd_clamp(epc, float(_W - 1))
            ribuf[pl.ds(sub * 16, 16)] = (rr * float(_W) + cc).astype(jnp.int32)
        pltpu.async_copy(imgs.at[ribuf], refbuf, rsem)
        for sub in range(_SUB):
            sb = sub * 16
            prev0 = vtb[pl.ds(sb, 16)]

            def it(i, c2, _sb=sb):
                prev, kmin = c2
                s0 = i * 8
                vv = [vtb[pl.ds((s0 + k) * 128 + _sb, 16)] for k in range(8)]
                vv.append(vtb[pl.ds(jnp.minimum(s0 + 8, _S - 1) * 128 + _sb,
                                    16)])
                for k in range(8):
                    s = s0 + k
                    pk_ = prev if k == 0 else vv[k - 1]
                    ismin = (vv[k] <= pk_) & (vv[k] <= vv[k + 1])
                    ks = jnp.abs(2 * s - (_S - 1)) * _S + s
                    kmin = jnp.minimum(
                        kmin,
                        jnp.where(ismin, jnp.broadcast_to(ks, (16,)), bigk))
                return (vv[7], kmin)

            _, kmin = lax.fori_loop(0, _S // 8, it, (prev0, bigk))
            if sub == 0:
                pltpu.make_async_copy(img_hbm.at[pl.ds(0, 128)], refbuf,
                                      rsem).wait()
            s_star = jnp.bitwise_and(kmin, _S - 1)
            w = jnp.where(kmin < bigk, 1.0, 0.0).astype(jnp.float32)
            vstar = plsc.load_gather(vtb, [s_star * 128 + (lane16 + sb)])
            tstar = plsc.load_gather(tsbuf, [s_star])
            epr = pkb[pl.ds(sub * 64, 16)]
            epc = pkb[pl.ds(sub * 64 + 16, 16)]
            nur = pkb[pl.ds(sub * 64 + 32, 16)]
            nuc = pkb[pl.ds(sub * 64 + 48, 16)]
            dr = ((epr + tstar * nur) - epr) * w
            dc = ((epc + tstar * nuc) - epc) * w
            plsc.store_scatter(dbuf, [lane16 + sub * 16, lane16 * 0], dr)
            plsc.store_scatter(dbuf, [lane16 + sub * 16, lane16 * 0 + 1], dc)
            refv = refbuf[pl.ds(sub * 16, 16)]
            d = refv - vstar
            acc_sq = acc_sq + w * d * d
            acc_w = acc_w + w
        pltpu.sync_copy(dbuf,
                        dist_hbm.at[pl.ds(wid * _PPW + sg * 128, 128), :])
        return (acc_sq, acc_w)

    issue_pk(0, pk0, psem0)
    drain_pk(pk0, psem0)
    comp_issue_rows(pk0, ir0, vt0, gsem0)
    issue_pk(jnp.int32(1), pk1, psem1)

    def outer(o, carry):
        sg = o * 2
        drain_pk(pk1, psem1)
        comp_issue_rows(pk1, ir1, vt1, gsem1)
        drain_rows(vt0, gsem0)
        carry = process(sg, pk0, vt0, carry)
        issue_pk(jnp.minimum(sg + 2, last), pk0, psem0)
        drain_pk(pk0, psem0)
        comp_issue_rows(pk0, ir0, vt0, gsem0)
        drain_rows(vt1, gsem1)
        carry = process(sg + 1, pk1, vt1, carry)
        issue_pk(jnp.minimum(sg + 3, last), pk1, psem1)
        return carry

    acc_sq, acc_w = lax.fori_loop(
        0, _SG // 2, outer,
        (jnp.zeros((16,), jnp.float32), jnp.zeros((16,), jnp.float32)))
    drain_rows(vt0, gsem0)
    drain_pk(pk1, psem1)
    accbuf[...] = acc_sq
    pltpu.sync_copy(accbuf, psq_hbm.at[pl.ds(wid * 16, 16)])
    accbuf[...] = acc_w
    pltpu.sync_copy(accbuf, pw_hbm.at[pl.ds(wid * 16, 16)])


@functools.lru_cache(maxsize=1)
def _sc_kernel():
    @functools.partial(
        pl.kernel,
        mesh=plsc.VectorSubcoreMesh(core_axis_name="c", subcore_axis_name="s"),
        compiler_params=pltpu.CompilerParams(needs_layout_passes=False),
        out_type=[
            jax.ShapeDtypeStruct((_N, 2), jnp.float32),
            jax.ShapeDtypeStruct((_NW * 16,), jnp.float32),
            jax.ShapeDtypeStruct((_NW * 16,), jnp.float32),
        ],
        scratch_types=[
            pltpu.VMEM((512,), jnp.float32),
            pltpu.VMEM((512,), jnp.float32),
            pltpu.VMEM((_S * 128,), jnp.int32),
            pltpu.VMEM((_S * 128,), jnp.int32),
            pltpu.VMEM((_S * 128,), jnp.float32),
            pltpu.VMEM((_S * 128,), jnp.float32),
            pltpu.VMEM((128,), jnp.int32),
            pltpu.VMEM((128,), jnp.float32),
            pltpu.VMEM((_S + 16,), jnp.float32),
            pltpu.VMEM((128, 2), jnp.float32),
            pltpu.VMEM((16,), jnp.float32),
            pltpu.VMEM_SHARED((_H * _W,), jnp.float32),
            pltpu.SemaphoreType.DMA,
            pltpu.SemaphoreType.DMA,
            pltpu.SemaphoreType.DMA,
            pltpu.SemaphoreType.DMA,
            pltpu.SemaphoreType.DMA,
        ],
    )
    def sck(img_hbm, pk_hbm, ts_hbm,
            dist_hbm, psq_hbm, pw_hbm, *scratch):
        _sc_body(img_hbm, pk_hbm, ts_hbm,
                 dist_hbm, psq_hbm, pw_hbm, *scratch)
    return sck


def kernel(edge_segments, distance_image):
    img = distance_image[0, 0]
    start = edge_segments[:, 0, :]
    end = edge_segments[:, 1, :]
    direction = end - start
    normals = jnp.stack([-direction[:, 1], direction[:, 0]], axis=-1)
    n_unit = normals / (jnp.linalg.norm(normals, axis=-1, keepdims=True) + 1e-8)
    t = jnp.linspace(0.0, 1.0, _P + 2)[1:-1]
    ts = jnp.linspace(-_EXT, _EXT, _S).astype(jnp.float32)

    sr = edge_segments[:, 0, 0]
    sc_ = edge_segments[:, 0, 1]
    er = edge_segments[:, 1, 0]
    ec = edge_segments[:, 1, 1]
    epr_mp = (1.0 - t)[None, :] * sr[:, None] + t[None, :] * er[:, None]
    epc_mp = (1.0 - t)[None, :] * sc_[:, None] + t[None, :] * ec[:, None]
    nur_mp = jnp.broadcast_to(n_unit[:, 0:1], (_M, _P))
    nuc_mp = jnp.broadcast_to(n_unit[:, 1:2], (_M, _P))

    pk = jnp.concatenate([epr_mp, epc_mp, nur_mp, nuc_mp], axis=1).reshape(-1)
    dist, psq, pw = _sc_kernel()(img.reshape(-1), pk, ts)
    losses = jnp.sum(psq) / jnp.maximum(jnp.sum(pw), 1.0)
    return losses, dist
